# Initial kernel scaffold; baseline (speedup 1.0000x reference)
import functools

import jax
import jax.numpy as jnp
from jax import lax
from jax.experimental import pallas as pl
from jax.experimental.pallas import tpu as pltpu
from jax.experimental.pallas import tpu_sc as plsc

_N = 10000
_E = 320000
_D = 128
_NC = 2
_NS = 16
_NW = _NC * _NS
_EPW = _E // _NW
_K = 80
_NCH = _EPW // _K
_STRIPE = 624
_LAST = _N - (_NS - 1) * _STRIPE
_DEGW = 16

_sc_mesh = plsc.VectorSubcoreMesh(core_axis_name="c", subcore_axis_name="s")


def _stripe_io(s, copy_fn_main, copy_fn_last):
    @pl.when(s < _NS - 1)
    def _():
        copy_fn_main()

    @pl.when(s == _NS - 1)
    def _():
        copy_fn_last()


@functools.partial(
    pl.kernel,
    mesh=_sc_mesh,
    out_type=jax.ShapeDtypeStruct((_NC, _N, _DEGW), jnp.float32),
    scratch_types=[
        pltpu.VMEM((_NCH, _K), jnp.int32),
        pltpu.VMEM((_K, _DEGW), jnp.float32),
        pltpu.VMEM_SHARED((_N, _DEGW), jnp.float32),
    ],
)
def _sc_degree(dst_hbm, zeros_hbm, ones_hbm, out_hbm, didx_v, ones_v, deg_sh):
    c = lax.axis_index("c")
    s = lax.axis_index("s")
    wid = c * _NS + s
    pltpu.sync_copy(ones_hbm, ones_v)
    pltpu.sync_copy(dst_hbm.at[wid], didx_v)
    base = s * _STRIPE
    _stripe_io(
        s,
        lambda: pltpu.sync_copy(zeros_hbm.at[pl.ds(0, _STRIPE)],
                                deg_sh.at[pl.ds(base, _STRIPE)]),
        lambda: pltpu.sync_copy(zeros_hbm, deg_sh.at[pl.ds(base, _LAST)]),
    )
    plsc.subcore_barrier()

    def body(j, carry):
        pltpu.sync_copy(ones_v, deg_sh.at[didx_v.at[j]], add=True)
        return carry

    lax.fori_loop(0, _NCH, body, 0)
    plsc.subcore_barrier()
    _stripe_io(
        s,
        lambda: pltpu.sync_copy(deg_sh.at[pl.ds(base, _STRIPE)],
                                out_hbm.at[c, pl.ds(base, _STRIPE)]),
        lambda: pltpu.sync_copy(deg_sh.at[pl.ds(base, _LAST)],
                                out_hbm.at[c, pl.ds(base, _LAST)]),
    )


@functools.partial(
    pl.kernel,
    mesh=_sc_mesh,
    out_type=jax.ShapeDtypeStruct((_NC, _N, _D), jnp.float32),
    scratch_types=[
        pltpu.VMEM((_NCH, _K), jnp.int32),
        pltpu.VMEM((_NCH, _K), jnp.int32),
        pltpu.VMEM((_K, _D), jnp.float32),
        pltpu.VMEM_SHARED((_N, _D), jnp.float32),
        pltpu.SemaphoreType.DMA,
    ],
)
def _sc_scatter(hs_hbm, src_hbm, dst_hbm, zeros_hbm, out_hbm,
                sidx_v, didx_v, rows_v, acc_sh, sem):
    c = lax.axis_index("c")
    s = lax.axis_index("s")
    wid = c * _NS + s
    pltpu.sync_copy(src_hbm.at[wid], sidx_v)
    pltpu.sync_copy(dst_hbm.at[wid], didx_v)
    base = s * _STRIPE
    _stripe_io(
        s,
        lambda: pltpu.sync_copy(zeros_hbm.at[pl.ds(0, _STRIPE)],
                                acc_sh.at[pl.ds(base, _STRIPE)]),
        lambda: pltpu.sync_copy(zeros_hbm, acc_sh.at[pl.ds(base, _LAST)]),
    )
    plsc.subcore_barrier()

    def body(j, carry):
        pltpu.async_copy(hs_hbm.at[sidx_v.at[j]], rows_v, sem).wait()
        pltpu.sync_copy(rows_v, acc_sh.at[didx_v.at[j]], add=True)
        return carry

    lax.fori_loop(0, _NCH, body, 0)
    plsc.subcore_barrier()
    _stripe_io(
        s,
        lambda: pltpu.sync_copy(acc_sh.at[pl.ds(base, _STRIPE)],
                                out_hbm.at[c, pl.ds(base, _STRIPE)]),
        lambda: pltpu.sync_copy(acc_sh.at[pl.ds(base, _LAST)],
                                out_hbm.at[c, pl.ds(base, _LAST)]),
    )


_B = 2000


def _dinv_block(degp_ref):
    deg = 1.0 + degp_ref[0] + degp_ref[1]
    return lax.rsqrt(deg)[:, 0:1]


def _tc_mm_scale_body(degp_ref, x_ref, w_ref, out_ref):
    out_ref[...] = jnp.dot(x_ref[...], w_ref[...],
                           preferred_element_type=jnp.float32) * _dinv_block(degp_ref)


_tc_mm_scale = pl.pallas_call(
    _tc_mm_scale_body,
    grid=(_N // _B,),
    in_specs=[
        pl.BlockSpec((_NC, _B, _DEGW), lambda i: (0, i, 0)),
        pl.BlockSpec((_B, _D), lambda i: (i, 0)),
        pl.BlockSpec((_D, _D), lambda i: (0, 0)),
    ],
    out_specs=pl.BlockSpec((_B, _D), lambda i: (i, 0)),
    out_shape=jax.ShapeDtypeStruct((_N, _D), jnp.float32),
)


def _tc_mid_body(degp_ref, acc_ref, hs_ref, b_ref, w_ref, out_ref):
    dinv = _dinv_block(degp_ref)
    pre = (acc_ref[0] + acc_ref[1] + hs_ref[...]) * dinv + b_ref[...]
    x2 = jnp.maximum(pre, 0.0)
    out_ref[...] = jnp.dot(x2, w_ref[...],
                           preferred_element_type=jnp.float32) * dinv


_tc_mid = pl.pallas_call(
    _tc_mid_body,
    grid=(_N // _B,),
    in_specs=[
        pl.BlockSpec((_NC, _B, _DEGW), lambda i: (0, i, 0)),
        pl.BlockSpec((_NC, _B, _D), lambda i: (0, i, 0)),
        pl.BlockSpec((_B, _D), lambda i: (i, 0)),
        pl.BlockSpec((1, _D), lambda i: (0, 0)),
        pl.BlockSpec((_D, _D), lambda i: (0, 0)),
    ],
    out_specs=pl.BlockSpec((_B, _D), lambda i: (i, 0)),
    out_shape=jax.ShapeDtypeStruct((_N, _D), jnp.float32),
)


def _tc_final_body(degp_ref, acc_ref, hs_ref, b_ref, out_ref):
    dinv = _dinv_block(degp_ref)
    out_ref[...] = (acc_ref[0] + acc_ref[1] + hs_ref[...]) * dinv + b_ref[...]


_tc_final = pl.pallas_call(
    _tc_final_body,
    grid=(_N // _B,),
    in_specs=[
        pl.BlockSpec((_NC, _B, _DEGW), lambda i: (0, i, 0)),
        pl.BlockSpec((_NC, _B, _D), lambda i: (0, i, 0)),
        pl.BlockSpec((_B, _D), lambda i: (i, 0)),
        pl.BlockSpec((1, _D), lambda i: (0, 0)),
    ],
    out_specs=pl.BlockSpec((_B, _D), lambda i: (i, 0)),
    out_shape=jax.ShapeDtypeStruct((_N, _D), jnp.float32),
)


def kernel(x, edge_index, batch, W1, b1, W2, b2):
    src3 = edge_index[0].reshape(_NW, _NCH, _K)
    dst3 = edge_index[1].reshape(_NW, _NCH, _K)
    zeros_deg = jnp.zeros((_LAST, _DEGW), jnp.float32)
    ones_deg = jnp.ones((_K, _DEGW), jnp.float32)
    zeros_acc = jnp.zeros((_LAST, _D), jnp.float32)

    degp = _sc_degree(dst3, zeros_deg, ones_deg)
    hs1 = _tc_mm_scale(degp, x, W1)
    acc1 = _sc_scatter(hs1, src3, dst3, zeros_acc)
    hs2 = _tc_mid(degp, acc1, hs1, b1.reshape(1, _D), W2)
    acc2 = _sc_scatter(hs2, src3, dst3, zeros_acc)
    out = _tc_final(degp, acc2, hs2, b2.reshape(1, _D))
    return out


# speedup vs baseline: 17.9316x; 17.9316x over previous
import functools

import jax
import jax.numpy as jnp
from jax import lax
from jax.experimental import pallas as pl
from jax.experimental.pallas import tpu as pltpu
from jax.experimental.pallas import tpu_sc as plsc

_N = 10000
_E = 320000
_D = 128
_NC = 2
_NS = 16
_NW = _NC * _NS
_EPW = _E // _NW
_K = 80
_NCH = _EPW // _K
_STRIPE = 624
_LAST = _N - (_NS - 1) * _STRIPE
_DEGW = 128

_sc_mesh = plsc.VectorSubcoreMesh(core_axis_name="c", subcore_axis_name="s")


def _stripe_io(s, copy_fn_main, copy_fn_last):
    @pl.when(s < _NS - 1)
    def _():
        copy_fn_main()

    @pl.when(s == _NS - 1)
    def _():
        copy_fn_last()


@functools.partial(
    pl.kernel,
    mesh=_sc_mesh,
    out_type=jax.ShapeDtypeStruct((_NC, _N, _DEGW), jnp.float32),
    scratch_types=[
        pltpu.VMEM((_NCH, _K), jnp.int32),
        pltpu.VMEM((_K, _DEGW), jnp.float32),
        pltpu.VMEM_SHARED((_N, _DEGW), jnp.float32),
    ],
)
def _sc_degree(dst_hbm, zeros_hbm, ones_hbm, out_hbm, didx_v, ones_v, deg_sh):
    c = lax.axis_index("c")
    s = lax.axis_index("s")
    wid = c * _NS + s
    pltpu.sync_copy(ones_hbm, ones_v)
    pltpu.sync_copy(dst_hbm.at[wid], didx_v)
    base = s * _STRIPE
    _stripe_io(
        s,
        lambda: pltpu.sync_copy(zeros_hbm.at[pl.ds(0, _STRIPE)],
                                deg_sh.at[pl.ds(base, _STRIPE)]),
        lambda: pltpu.sync_copy(zeros_hbm, deg_sh.at[pl.ds(base, _LAST)]),
    )
    plsc.subcore_barrier()

    def body(j, carry):
        pltpu.sync_copy(ones_v, deg_sh.at[didx_v.at[j]], add=True)
        return carry

    lax.fori_loop(0, _NCH, body, 0)
    plsc.subcore_barrier()
    _stripe_io(
        s,
        lambda: pltpu.sync_copy(deg_sh.at[pl.ds(base, _STRIPE)],
                                out_hbm.at[c, pl.ds(base, _STRIPE)]),
        lambda: pltpu.sync_copy(deg_sh.at[pl.ds(base, _LAST)],
                                out_hbm.at[c, pl.ds(base, _LAST)]),
    )


@functools.partial(
    pl.kernel,
    mesh=_sc_mesh,
    out_type=jax.ShapeDtypeStruct((_NC, _N, _D), jnp.float32),
    scratch_types=[
        pltpu.VMEM((_NCH, _K), jnp.int32),
        pltpu.VMEM((_NCH, _K), jnp.int32),
        pltpu.VMEM((_K, _D), jnp.float32),
        pltpu.VMEM_SHARED((_N, _D), jnp.float32),
        pltpu.SemaphoreType.DMA,
    ],
)
def _sc_scatter(hs_hbm, src_hbm, dst_hbm, zeros_hbm, out_hbm,
                sidx_v, didx_v, rows_v, acc_sh, sem):
    c = lax.axis_index("c")
    s = lax.axis_index("s")
    wid = c * _NS + s
    pltpu.sync_copy(src_hbm.at[wid], sidx_v)
    pltpu.sync_copy(dst_hbm.at[wid], didx_v)
    base = s * _STRIPE
    _stripe_io(
        s,
        lambda: pltpu.sync_copy(zeros_hbm.at[pl.ds(0, _STRIPE)],
                                acc_sh.at[pl.ds(base, _STRIPE)]),
        lambda: pltpu.sync_copy(zeros_hbm, acc_sh.at[pl.ds(base, _LAST)]),
    )
    plsc.subcore_barrier()

    def body(j, carry):
        pltpu.async_copy(hs_hbm.at[sidx_v.at[j]], rows_v, sem).wait()
        pltpu.sync_copy(rows_v, acc_sh.at[didx_v.at[j]], add=True)
        return carry

    lax.fori_loop(0, _NCH, body, 0)
    plsc.subcore_barrier()
    _stripe_io(
        s,
        lambda: pltpu.sync_copy(acc_sh.at[pl.ds(base, _STRIPE)],
                                out_hbm.at[c, pl.ds(base, _STRIPE)]),
        lambda: pltpu.sync_copy(acc_sh.at[pl.ds(base, _LAST)],
                                out_hbm.at[c, pl.ds(base, _LAST)]),
    )


_B = 2000


def _dinv_block(degp_ref):
    deg = 1.0 + degp_ref[0] + degp_ref[1]
    return lax.rsqrt(deg)[:, 0:1]


def _tc_mm_scale_body(degp_ref, x_ref, w_ref, out_ref):
    out_ref[...] = jnp.dot(x_ref[...], w_ref[...],
                           preferred_element_type=jnp.float32) * _dinv_block(degp_ref)


_tc_mm_scale = pl.pallas_call(
    _tc_mm_scale_body,
    grid=(_N // _B,),
    in_specs=[
        pl.BlockSpec((_NC, _B, _DEGW), lambda i: (0, i, 0)),
        pl.BlockSpec((_B, _D), lambda i: (i, 0)),
        pl.BlockSpec((_D, _D), lambda i: (0, 0)),
    ],
    out_specs=pl.BlockSpec((_B, _D), lambda i: (i, 0)),
    out_shape=jax.ShapeDtypeStruct((_N, _D), jnp.float32),
)


def _tc_mid_body(degp_ref, acc_ref, hs_ref, b_ref, w_ref, out_ref):
    dinv = _dinv_block(degp_ref)
    pre = (acc_ref[0] + acc_ref[1] + hs_ref[...]) * dinv + b_ref[...]
    x2 = jnp.maximum(pre, 0.0)
    out_ref[...] = jnp.dot(x2, w_ref[...],
                           preferred_element_type=jnp.float32) * dinv


_tc_mid = pl.pallas_call(
    _tc_mid_body,
    grid=(_N // _B,),
    in_specs=[
        pl.BlockSpec((_NC, _B, _DEGW), lambda i: (0, i, 0)),
        pl.BlockSpec((_NC, _B, _D), lambda i: (0, i, 0)),
        pl.BlockSpec((_B, _D), lambda i: (i, 0)),
        pl.BlockSpec((1, _D), lambda i: (0, 0)),
        pl.BlockSpec((_D, _D), lambda i: (0, 0)),
    ],
    out_specs=pl.BlockSpec((_B, _D), lambda i: (i, 0)),
    out_shape=jax.ShapeDtypeStruct((_N, _D), jnp.float32),
)


def _tc_final_body(degp_ref, acc_ref, hs_ref, b_ref, out_ref):
    dinv = _dinv_block(degp_ref)
    out_ref[...] = (acc_ref[0] + acc_ref[1] + hs_ref[...]) * dinv + b_ref[...]


_tc_final = pl.pallas_call(
    _tc_final_body,
    grid=(_N // _B,),
    in_specs=[
        pl.BlockSpec((_NC, _B, _DEGW), lambda i: (0, i, 0)),
        pl.BlockSpec((_NC, _B, _D), lambda i: (0, i, 0)),
        pl.BlockSpec((_B, _D), lambda i: (i, 0)),
        pl.BlockSpec((1, _D), lambda i: (0, 0)),
    ],
    out_specs=pl.BlockSpec((_B, _D), lambda i: (i, 0)),
    out_shape=jax.ShapeDtypeStruct((_N, _D), jnp.float32),
)


def kernel(x, edge_index, batch, W1, b1, W2, b2):
    src3 = edge_index[0].reshape(_NW, _NCH, _K)
    dst3 = edge_index[1].reshape(_NW, _NCH, _K)
    zeros_deg = jnp.zeros((_LAST, _DEGW), jnp.float32)
    ones_deg = jnp.ones((_K, _DEGW), jnp.float32)
    zeros_acc = jnp.zeros((_LAST, _D), jnp.float32)

    degp = _sc_degree(dst3, zeros_deg, ones_deg)
    hs1 = _tc_mm_scale(degp, x, W1)
    acc1 = _sc_scatter(hs1, src3, dst3, zeros_acc)
    hs2 = _tc_mid(degp, acc1, hs1, b1.reshape(1, _D), W2)
    acc2 = _sc_scatter(hs2, src3, dst3, zeros_acc)
    out = _tc_final(degp, acc2, hs2, b2.reshape(1, _D))
    return out
